# baseline (speedup 1.0000x reference)
import functools

import jax
import jax.numpy as jnp
from jax import lax
from jax.experimental import pallas as pl
from jax.experimental.pallas import tpu as pltpu
from jax.experimental.pallas import tpu_sc as plsc

N = 10000
D = 128
FEA = 32
H = 8
C = 16
E1 = 320000
E2 = 160000
B = 128
L = 32

NC = 2
NS = 16
NW = NC * NS
RBLK = 80
NRB = N // RBLK

f32 = jnp.float32
i32 = jnp.int32


def _k1_body(poi, Wcat, M1, M2, h1_o, h2_o, ps1_o, pd1_o, ps2_o, pd2_o):
    hcat = jnp.dot(poi[...], Wcat[...], preferred_element_type=f32)
    h1 = hcat[:, :D]
    h2 = hcat[:, D:]
    h1_o[...] = h1
    h2_o[...] = h2
    p1 = jnp.dot(h1, M1[...], preferred_element_type=f32)
    p2 = jnp.dot(h2, M2[...], preferred_element_type=f32)
    ps1_o[...] = p1[:, :16]
    pd1_o[...] = p1[:, 16:]
    ps2_o[...] = p2[:, :16]
    pd2_o[...] = p2[:, 16:]


def _precompute(poi, Wcat, M1, M2):
    blk = 1000
    grid = (N // blk,)
    full = lambda shape: pl.BlockSpec(shape, lambda i: (0, 0))
    row = lambda w: pl.BlockSpec((blk, w), lambda i: (i, 0))
    return pl.pallas_call(
        _k1_body,
        grid=grid,
        in_specs=[row(D), full((D, 2 * D)), full((D, 32)), full((D, 32))],
        out_specs=[row(D), row(D), row(16), row(16), row(16), row(16)],
        out_shape=[
            jax.ShapeDtypeStruct((N, D), f32),
            jax.ShapeDtypeStruct((N, D), f32),
            jax.ShapeDtypeStruct((N, 16), f32),
            jax.ShapeDtypeStruct((N, 16), f32),
            jax.ShapeDtypeStruct((N, 16), f32),
            jax.ShapeDtypeStruct((N, 16), f32),
        ],
    )(poi, Wcat, M1, M2)


def _lane_bcast(v, lane):
    idx = jnp.full((16, 1), lane, i32)
    dn = lax.GatherDimensionNumbers(
        offset_dims=(), collapsed_slice_dims=(0,), start_index_map=(0,))
    return lax.gather(v, idx, dn, (1,),
                      mode=lax.GatherScatterMode.PROMISE_IN_BOUNDS)


NBUF = 4


def _edge_pass(hmat, psm, pdm, srcr, dstr, zn, zd, accn, accd, num_o, den_o,
               bufs, gsems, ssems, isems, epw, K):
    cid = lax.axis_index("c")
    tid = lax.axis_index("s")
    wid = cid * NS + tid

    def for_my_row_blocks(fn):
        for i in range((NRB + NS - 1) // NS):
            blk = i * NS + tid

            @pl.when(blk < NRB)
            def _():
                fn(blk * RBLK)

    def zero_blk(off):
        pltpu.sync_copy(zn, accn.at[pl.ds(off, RBLK)])
        pltpu.sync_copy(zd, accd.at[pl.ds(off, RBLK)])

    for_my_row_blocks(zero_blk)
    plsc.subcore_barrier()

    nch = epw // K
    nepi = nch % NBUF

    def fire_idx(c, b):
        base = wid * epw + c * K
        pltpu.async_copy(srcr.at[pl.ds(base, K)], bufs[b][0], isems[b])
        pltpu.async_copy(dstr.at[pl.ds(base, K)], bufs[b][1], isems[b])

    def wait_idx(c, b):
        base = wid * epw + c * K
        pltpu.make_async_copy(srcr.at[pl.ds(base, K)], bufs[b][0], isems[b]).wait()
        pltpu.make_async_copy(dstr.at[pl.ds(base, K)], bufs[b][1], isems[b]).wait()

    def fire_gathers(b):
        sidx, didx, psb, pdb, hsb, dvb = bufs[b]
        pltpu.async_copy(psm.at[sidx], psb, gsems[b])
        pltpu.async_copy(pdm.at[didx], pdb, gsems[b])
        pltpu.async_copy(hmat.at[sidx], hsb, gsems[b])

    def wait_gathers(b):
        sidx, didx, psb, pdb, hsb, dvb = bufs[b]
        pltpu.make_async_copy(psm.at[sidx], psb, gsems[b]).wait()
        pltpu.make_async_copy(pdm.at[didx], pdb, gsems[b]).wait()
        pltpu.make_async_copy(hmat.at[sidx], hsb, gsems[b]).wait()

    def fire_scatters(b):
        sidx, didx, psb, pdb, hsb, dvb = bufs[b]
        pltpu.async_copy(hsb, accn.at[didx], ssems[b], add=True)
        pltpu.async_copy(dvb, accd.at[didx], ssems[b], add=True)

    def wait_scatters(b):
        sidx, didx, psb, pdb, hsb, dvb = bufs[b]
        pltpu.make_async_copy(hsb, accn.at[didx], ssems[b]).wait()
        pltpu.make_async_copy(dvb, accd.at[didx], ssems[b]).wait()

    def compute(b):
        sidx, didx, psb, pdb, hsb, dvb = bufs[b]

        @plsc.parallel_loop(0, K, unroll=6)
        def edge(k):
            e16 = psb[k, :] + pdb[k, :]
            e16 = jnp.maximum(e16, 0.2 * e16)
            ex = jnp.exp(e16)
            dvb[k, :] = ex
            for h in range(H):
                exh = _lane_bcast(ex, h)
                hsb[k, pl.ds(h * 16, 16)] = hsb[k, pl.ds(h * 16, 16)] * exh

    def step(c, j):
        b = j
        b1 = (j + 1) % NBUF
        b2 = (j + 2) % NBUF
        cond_pref = c + 2 < nch

        if isinstance(c, int):
            if cond_pref:
                if c >= 2:
                    wait_scatters(b2)
                fire_idx(c + 2, b2)
            if c + 1 < nch:
                wait_idx(c + 1, b1)
                fire_gathers(b1)
        else:
            @pl.when(cond_pref)
            def _():
                if j >= 2:
                    wait_scatters(b2)
                else:
                    @pl.when(c >= 2)
                    def _():
                        wait_scatters(b2)
                fire_idx(c + 2, b2)

            @pl.when(c + 1 < nch)
            def _():
                wait_idx(c + 1, b1)
                fire_gathers(b1)

        wait_gathers(b)
        compute(b)
        fire_scatters(b)

    fire_idx(0, 0)
    wait_idx(0, 0)
    fire_gathers(0)
    fire_idx(1, 1)

    def body(i, carry):
        for j in range(NBUF):
            step(NBUF * i + j, j)
        return carry

    lax.fori_loop(0, nch // NBUF, body, 0)

    for ce in range(nch - nepi, nch):
        step(ce, ce % NBUF)
    for b in range(NBUF):
        wait_scatters(b)
    plsc.subcore_barrier()

    def copyout_blk(off):
        pltpu.sync_copy(accn.at[pl.ds(off, RBLK)],
                        num_o.at[cid, pl.ds(off, RBLK)])
        pltpu.sync_copy(accd.at[pl.ds(off, RBLK)],
                        den_o.at[cid, pl.ds(off, RBLK)])

    for_my_row_blocks(copyout_blk)


KE = 40


def _gat_edges(h1, ps1, pd1, h2, ps2, pd2, src1, dst1, src2, dst2, zn, zd):
    mesh = plsc.VectorSubcoreMesh(core_axis_name="c", subcore_axis_name="s",
                                  num_cores=NC, num_subcores=NS)

    @functools.partial(
        pl.kernel,
        out_type=[
            jax.ShapeDtypeStruct((NC, N, D), f32),
            jax.ShapeDtypeStruct((NC, N, 16), f32),
            jax.ShapeDtypeStruct((NC, N, D), f32),
            jax.ShapeDtypeStruct((NC, N, 16), f32),
        ],
        mesh=mesh,
        scratch_types=(
            [pltpu.VMEM_SHARED((N, D), f32), pltpu.VMEM_SHARED((N, 16), f32)]
            + [pltpu.VMEM((KE,), i32), pltpu.VMEM((KE,), i32),
               pltpu.VMEM((KE, 16), f32), pltpu.VMEM((KE, 16), f32),
               pltpu.VMEM((KE, D), f32), pltpu.VMEM((KE, 16), f32)] * NBUF
            + [pltpu.SemaphoreType.DMA] * (3 * NBUF)
        ),
        compiler_params=pltpu.CompilerParams(use_tc_tiling_on_sc=False),
    )
    def k(h1r, ps1r, pd1r, h2r, ps2r, pd2r, s1r, d1r, s2r, d2r, znr, zdr,
          n1_o, d1_o, n2_o, d2_o, accn, accd, *scr):
        bufs = [tuple(scr[6 * b:6 * b + 6]) for b in range(NBUF)]
        sems = scr[6 * NBUF:]
        gsems = sems[:NBUF]
        ssems = sems[NBUF:2 * NBUF]
        isems = sems[2 * NBUF:]
        _edge_pass(h1r, ps1r, pd1r, s1r, d1r, znr, zdr, accn, accd,
                   n1_o, d1_o, bufs, gsems, ssems, isems, E1 // NW, KE)
        _edge_pass(h2r, ps2r, pd2r, s2r, d2r, znr, zdr, accn, accd,
                   n2_o, d2_o, bufs, gsems, ssems, isems, E2 // NW, KE)

    return k(h1, ps1, pd1, h2, ps2, pd2, src1, dst1, src2, dst2, zn, zd)


def _k3_body(n1, d1, n2, d2, fea, bg1, bg2, Wf, bf, W1a, b1a, W1b, b1b,
             W2a, b2a, W2b, b2b, Rm, emb_o):
    num1 = n1[0] + n1[1]
    den1 = jnp.dot(d1[0] + d1[1], Rm[...], preferred_element_type=f32)
    w1 = num1 / (den1 + 1e-16) + bg1[...]
    num2 = n2[0] + n2[1]
    den2 = jnp.dot(d2[0] + d2[1], Rm[...], preferred_element_type=f32)
    w2 = num2 / (den2 + 1e-16) + bg2[...]

    fx = fea[...]
    nrm = jnp.maximum(jnp.sqrt(jnp.sum(fx * fx, axis=-1, keepdims=True)), 1e-12)
    fn = fx / nrm
    w3 = jnp.maximum(jnp.dot(fn, Wf[...], preferred_element_type=f32) + bf[...], 0.0)

    W1a_v = W1a[...]
    e1 = jnp.maximum(
        jnp.dot(w1, W1a_v[:D], preferred_element_type=f32)
        + jnp.dot(w2, W1a_v[D:], preferred_element_type=f32) + b1a[...], 0.0)
    emb1 = jnp.dot(e1, W1b[...], preferred_element_type=f32) + b1b[...]
    W2a_v = W2a[...]
    e2 = jnp.maximum(
        jnp.dot(emb1, W2a_v[:D], preferred_element_type=f32)
        + jnp.dot(w3, W2a_v[D:], preferred_element_type=f32) + b2a[...], 0.0)
    emb_o[...] = jnp.dot(e2, W2b[...], preferred_element_type=f32) + b2b[...]


def _combine_mlp(n1, d1, n2, d2, fea, bg1, bg2, Wf, bf, W1a, b1a, W1b, b1b,
                 W2a, b2a, W2b, b2b, Rm):
    blk = 1000
    grid = (N // blk,)
    full = lambda shape: pl.BlockSpec(shape, lambda i: tuple(0 for _ in shape))
    part_n = pl.BlockSpec((NC, blk, D), lambda i: (0, i, 0))
    part_d = pl.BlockSpec((NC, blk, 16), lambda i: (0, i, 0))
    row = lambda w: pl.BlockSpec((blk, w), lambda i: (i, 0))
    return pl.pallas_call(
        _k3_body,
        grid=grid,
        in_specs=[part_n, part_d, part_n, part_d, row(FEA),
                  full((1, D)), full((1, D)), full((FEA, D)), full((1, D)),
                  full((2 * D, D)), full((1, D)), full((D, D)), full((1, D)),
                  full((2 * D, D)), full((1, D)), full((D, D)), full((1, D)),
                  full((16, D))],
        out_specs=row(D),
        out_shape=jax.ShapeDtypeStruct((N, D), f32),
    )(n1, d1, n2, d2, fea, bg1, bg2, Wf, bf, W1a, b1a, W1b, b1b,
      W2a, b2a, W2b, b2b, Rm)


def _batch_gather(emb, bidx_t):
    nrows = B * L
    per_w = nrows // NW
    mesh = plsc.VectorSubcoreMesh(core_axis_name="c", subcore_axis_name="s",
                                  num_cores=NC, num_subcores=NS)

    @functools.partial(
        pl.kernel,
        out_type=jax.ShapeDtypeStruct((nrows, D), f32),
        mesh=mesh,
        scratch_types=[
            pltpu.VMEM((per_w,), i32),
            pltpu.VMEM((per_w, D), f32),
            pltpu.SemaphoreType.DMA,
        ],
    )
    def k(emb_r, bidx_r, out_r, idxv, rows, sem):
        wid = lax.axis_index("c") * NS + lax.axis_index("s")
        base = wid * per_w
        pltpu.sync_copy(bidx_r.at[pl.ds(base, per_w)], idxv)
        pltpu.async_copy(emb_r.at[idxv], rows, sem).wait()
        pltpu.sync_copy(rows, out_r.at[pl.ds(base, per_w)])

    return k(emb, bidx_t)


def _sig(x):
    return 1.0 / (1.0 + jnp.exp(-x))


def _k5_body(xb, WihT0, WhhT0, bih0, bhh0, WihT1, WhhT1, bih1, bhh1,
             out_o, gx_ref, ys_ref, hc2_ref):
    xb_v = xb[...]

    def run_layer(x_flat, WihT, WhhT, bias):
        gx = jnp.dot(x_flat, WihT, preferred_element_type=f32) + bias
        gx_ref[...] = gx.reshape(L, B, 4 * D)
        z = jnp.zeros((B, D), f32)

        def step(t, carry):
            h, c = carry
            g = gx_ref[t] + jnp.dot(h, WhhT, preferred_element_type=f32)
            ii = _sig(g[:, :D])
            ff = _sig(g[:, D:2 * D])
            gg = jnp.tanh(g[:, 2 * D:3 * D])
            oo = _sig(g[:, 3 * D:])
            c2 = ff * c + ii * gg
            h2 = oo * jnp.tanh(c2)
            ys_ref[t] = h2
            return (h2, c2)

        lax.fori_loop(0, L, step, (z, z))

    def softmax_stats():
        def mx(t, m):
            return jnp.maximum(m, ys_ref[t])
        m = lax.fori_loop(0, L, mx, jnp.full((B, D), -jnp.inf, f32))

        def sm(t, s):
            return s + jnp.exp(ys_ref[t] - m)
        s = lax.fori_loop(0, L, sm, jnp.zeros((B, D), f32))
        return m, s

    run_layer(xb_v, WihT0[...], WhhT0[...], bih0[...] + bhh0[...])
    m, s = softmax_stats()

    def apply1(t, carry):
        yt = ys_ref[t]
        at = jnp.mean(jnp.exp(yt - m) / s, axis=-1, keepdims=True)
        o1 = yt * (1.0 + at)
        hc2_ref[t] = xb[t] + jnp.maximum(o1, 0.0)
        return carry

    lax.fori_loop(0, L, apply1, 0)

    run_layer(hc2_ref[...].reshape(L * B, D), WihT1[...], WhhT1[...],
              bih1[...] + bhh1[...])
    m2, s2 = softmax_stats()
    y_last = ys_ref[L - 1]
    a_last = jnp.mean(jnp.exp(y_last - m2) / s2, axis=-1, keepdims=True)
    out_o[...] = y_last * (1.0 + a_last)


def _lstm_head(xb, WihT0, WhhT0, bih0, bhh0, WihT1, WhhT1, bih1, bhh1):
    xb3 = xb.reshape(L, B, D)
    vm = pl.BlockSpec(memory_space=pltpu.MemorySpace.VMEM)
    return pl.pallas_call(
        _k5_body,
        in_specs=[vm] * 9,
        out_specs=vm,
        out_shape=jax.ShapeDtypeStruct((B, D), f32),
        scratch_shapes=[
            pltpu.VMEM((L, B, 4 * D), f32),
            pltpu.VMEM((L, B, D), f32),
            pltpu.VMEM((L, B, D), f32),
        ],
    )(xb3, WihT0, WhhT0, bih0, bhh0, WihT1, WhhT1, bih1, bhh1)


def kernel(poi_features, fea_x, edge_index, str_edge_index, batch_x,
           W_g1, a_src1, a_dst1, b_g1, W_g2, a_src2, a_dst2, b_g2,
           Wf, bf, W1a, b1a, W1b, b1b, W2a, b2a, W2b, b2b,
           Wih0, Whh0, bih0, bhh0, Wih1, Whh1, bih1, bhh1):
    Wcat = jnp.concatenate([W_g1, W_g2], axis=1)
    sel = jnp.tile(jnp.eye(H, dtype=f32), (1, 2))

    def mk(a):
        return (a[:, :, None] * sel[:, None, :]).reshape(H * C, 16)

    M1 = jnp.concatenate([mk(a_src1), mk(a_dst1)], axis=1)
    M2 = jnp.concatenate([mk(a_src2), mk(a_dst2)], axis=1)
    Rm = (jnp.arange(16)[:, None] == (jnp.arange(D) // C)[None, :]).astype(f32)
    zn = jnp.zeros((RBLK, D), f32)
    zd = jnp.zeros((RBLK, 16), f32)
    r1 = lambda b: b.reshape(1, -1)

    h1, h2, ps1, pd1, ps2, pd2 = _precompute(poi_features, Wcat, M1, M2)

    n1, d1, n2, d2 = _gat_edges(
        h1, ps1, pd1, h2, ps2, pd2,
        edge_index[0], edge_index[1],
        str_edge_index[0], str_edge_index[1], zn, zd)

    emb = _combine_mlp(n1, d1, n2, d2, fea_x, r1(b_g1), r1(b_g2), Wf, r1(bf),
                       W1a, r1(b1a), W1b, r1(b1b), W2a, r1(b2a), W2b, r1(b2b),
                       Rm)

    bidx_t = jnp.transpose(batch_x).reshape(-1).astype(i32)
    xb = _batch_gather(emb, bidx_t)

    return _lstm_head(xb, Wih0.T, Whh0.T, r1(bih0), r1(bhh0),
                      Wih1.T, Whh1.T, r1(bih1), r1(bhh1))


# speedup vs baseline: 1.2065x; 1.2065x over previous
import functools

import jax
import jax.numpy as jnp
from jax import lax
from jax.experimental import pallas as pl
from jax.experimental.pallas import tpu as pltpu
from jax.experimental.pallas import tpu_sc as plsc

N = 10000
D = 128
FEA = 32
H = 8
C = 16
E1 = 320000
E2 = 160000
B = 128
L = 32

NC = 2
NS = 16
NW = NC * NS
RBLK = 80
NRB = N // RBLK

f32 = jnp.float32
i32 = jnp.int32


def _k1_body(poi, Wcat, M1, M2, h1_o, h2_o, ps1_o, pd1_o, ps2_o, pd2_o):
    hcat = jnp.dot(poi[...], Wcat[...], preferred_element_type=f32)
    h1 = hcat[:, :D]
    h2 = hcat[:, D:]
    h1_o[...] = h1
    h2_o[...] = h2
    p1 = jnp.dot(h1, M1[...], preferred_element_type=f32)
    p2 = jnp.dot(h2, M2[...], preferred_element_type=f32)
    ps1_o[...] = p1[:, :16]
    pd1_o[...] = p1[:, 16:]
    ps2_o[...] = p2[:, :16]
    pd2_o[...] = p2[:, 16:]


def _precompute(poi, Wcat, M1, M2):
    blk = 1000
    grid = (N // blk,)
    full = lambda shape: pl.BlockSpec(shape, lambda i: (0, 0))
    row = lambda w: pl.BlockSpec((blk, w), lambda i: (i, 0))
    return pl.pallas_call(
        _k1_body,
        grid=grid,
        in_specs=[row(D), full((D, 2 * D)), full((D, 32)), full((D, 32))],
        out_specs=[row(D), row(D), row(16), row(16), row(16), row(16)],
        out_shape=[
            jax.ShapeDtypeStruct((N, D), f32),
            jax.ShapeDtypeStruct((N, D), f32),
            jax.ShapeDtypeStruct((N, 16), f32),
            jax.ShapeDtypeStruct((N, 16), f32),
            jax.ShapeDtypeStruct((N, 16), f32),
            jax.ShapeDtypeStruct((N, 16), f32),
        ],
    )(poi, Wcat, M1, M2)


def _lane_bcast(v, lane):
    idx = jnp.full((16, 1), lane, i32)
    dn = lax.GatherDimensionNumbers(
        offset_dims=(), collapsed_slice_dims=(0,), start_index_map=(0,))
    return lax.gather(v, idx, dn, (1,),
                      mode=lax.GatherScatterMode.PROMISE_IN_BOUNDS)


NBUF = 4


def _edge_pass(hmat, psm, pdm, srcr, dstr, zn, zd, accn, accd, num_o, den_o,
               bufs, gsems, ssems, isems, epw, K):
    cid = lax.axis_index("c")
    tid = lax.axis_index("s")
    wid = cid * NS + tid

    def for_my_row_blocks(fn):
        for i in range((NRB + NS - 1) // NS):
            blk = i * NS + tid

            @pl.when(blk < NRB)
            def _():
                fn(blk * RBLK)

    def zero_blk(off):
        pltpu.sync_copy(zn, accn.at[pl.ds(off, RBLK)])
        pltpu.sync_copy(zd, accd.at[pl.ds(off, RBLK)])

    for_my_row_blocks(zero_blk)
    plsc.subcore_barrier()

    nch = epw // K
    nepi = nch % NBUF

    def fire_idx(c, b):
        base = wid * epw + c * K
        pltpu.async_copy(srcr.at[pl.ds(base, K)], bufs[b][0], isems[b])
        pltpu.async_copy(dstr.at[pl.ds(base, K)], bufs[b][1], isems[b])

    def wait_idx(c, b):
        base = wid * epw + c * K
        pltpu.make_async_copy(srcr.at[pl.ds(base, K)], bufs[b][0], isems[b]).wait()
        pltpu.make_async_copy(dstr.at[pl.ds(base, K)], bufs[b][1], isems[b]).wait()

    def fire_gathers(b):
        sidx, didx, psb, pdb, hsb, dvb = bufs[b]
        pltpu.async_copy(psm.at[sidx], psb, gsems[b])
        pltpu.async_copy(pdm.at[didx], pdb, gsems[b])
        pltpu.async_copy(hmat.at[sidx], hsb, gsems[b])

    def wait_gathers(b):
        sidx, didx, psb, pdb, hsb, dvb = bufs[b]
        pltpu.make_async_copy(psm.at[sidx], psb, gsems[b]).wait()
        pltpu.make_async_copy(pdm.at[didx], pdb, gsems[b]).wait()
        pltpu.make_async_copy(hmat.at[sidx], hsb, gsems[b]).wait()

    def fire_scatters(b):
        sidx, didx, psb, pdb, hsb, dvb = bufs[b]
        pltpu.async_copy(hsb, accn.at[didx], ssems[b], add=True)
        pltpu.async_copy(dvb, accd.at[didx], ssems[b], add=True)

    def wait_scatters(b):
        sidx, didx, psb, pdb, hsb, dvb = bufs[b]
        pltpu.make_async_copy(hsb, accn.at[didx], ssems[b]).wait()
        pltpu.make_async_copy(dvb, accd.at[didx], ssems[b]).wait()

    def compute(b):
        sidx, didx, psb, pdb, hsb, dvb = bufs[b]

        @plsc.parallel_loop(0, K, unroll=5)
        def edge(k):
            e16 = psb[k, :] + pdb[k, :]
            e16 = jnp.maximum(e16, 0.2 * e16)
            ex = jnp.exp(e16)
            dvb[k, :] = ex
            for h in range(H):
                exh = _lane_bcast(ex, h)
                hsb[k, pl.ds(h * 16, 16)] = hsb[k, pl.ds(h * 16, 16)] * exh

    def step(c, j):
        b = j
        b1 = (j + 1) % NBUF
        b2 = (j + 2) % NBUF
        cond_pref = c + 2 < nch

        if isinstance(c, int):
            if cond_pref:
                if c >= 2:
                    wait_scatters(b2)
                fire_idx(c + 2, b2)
            if c + 1 < nch:
                wait_idx(c + 1, b1)
                fire_gathers(b1)
        else:
            @pl.when(cond_pref)
            def _():
                if j >= 2:
                    wait_scatters(b2)
                else:
                    @pl.when(c >= 2)
                    def _():
                        wait_scatters(b2)
                fire_idx(c + 2, b2)

            @pl.when(c + 1 < nch)
            def _():
                wait_idx(c + 1, b1)
                fire_gathers(b1)

        wait_gathers(b)
        compute(b)
        fire_scatters(b)

    fire_idx(0, 0)
    wait_idx(0, 0)
    fire_gathers(0)
    fire_idx(1, 1)

    def body(i, carry):
        for j in range(NBUF):
            step(NBUF * i + j, j)
        return carry

    lax.fori_loop(0, nch // NBUF, body, 0)

    for ce in range(nch - nepi, nch):
        step(ce, ce % NBUF)
    for b in range(NBUF):
        wait_scatters(b)
    plsc.subcore_barrier()

    def copyout_blk(off):
        pltpu.sync_copy(accn.at[pl.ds(off, RBLK)],
                        num_o.at[cid, pl.ds(off, RBLK)])
        pltpu.sync_copy(accd.at[pl.ds(off, RBLK)],
                        den_o.at[cid, pl.ds(off, RBLK)])

    for_my_row_blocks(copyout_blk)


KE = 40


def _gat_edges(h1, ps1, pd1, h2, ps2, pd2, src1, dst1, src2, dst2, zn, zd):
    mesh = plsc.VectorSubcoreMesh(core_axis_name="c", subcore_axis_name="s",
                                  num_cores=NC, num_subcores=NS)

    @functools.partial(
        pl.kernel,
        out_type=[
            jax.ShapeDtypeStruct((NC, N, D), f32),
            jax.ShapeDtypeStruct((NC, N, 16), f32),
            jax.ShapeDtypeStruct((NC, N, D), f32),
            jax.ShapeDtypeStruct((NC, N, 16), f32),
        ],
        mesh=mesh,
        scratch_types=(
            [pltpu.VMEM_SHARED((N, D), f32), pltpu.VMEM_SHARED((N, 16), f32)]
            + [pltpu.VMEM((KE,), i32), pltpu.VMEM((KE,), i32),
               pltpu.VMEM((KE, 16), f32), pltpu.VMEM((KE, 16), f32),
               pltpu.VMEM((KE, D), f32), pltpu.VMEM((KE, 16), f32)] * NBUF
            + [pltpu.SemaphoreType.DMA] * (3 * NBUF)
        ),
        compiler_params=pltpu.CompilerParams(use_tc_tiling_on_sc=False),
    )
    def k(h1r, ps1r, pd1r, h2r, ps2r, pd2r, s1r, d1r, s2r, d2r, znr, zdr,
          n1_o, d1_o, n2_o, d2_o, accn, accd, *scr):
        bufs = [tuple(scr[6 * b:6 * b + 6]) for b in range(NBUF)]
        sems = scr[6 * NBUF:]
        gsems = sems[:NBUF]
        ssems = sems[NBUF:2 * NBUF]
        isems = sems[2 * NBUF:]
        _edge_pass(h1r, ps1r, pd1r, s1r, d1r, znr, zdr, accn, accd,
                   n1_o, d1_o, bufs, gsems, ssems, isems, E1 // NW, KE)
        _edge_pass(h2r, ps2r, pd2r, s2r, d2r, znr, zdr, accn, accd,
                   n2_o, d2_o, bufs, gsems, ssems, isems, E2 // NW, KE)

    return k(h1, ps1, pd1, h2, ps2, pd2, src1, dst1, src2, dst2, zn, zd)


def _k3_body(n1, d1, n2, d2, fea, bg1, bg2, Wf, bf, W1a, b1a, W1b, b1b,
             W2a, b2a, W2b, b2b, Rm, emb_o):
    num1 = n1[0] + n1[1]
    den1 = jnp.dot(d1[0] + d1[1], Rm[...], preferred_element_type=f32)
    w1 = num1 / (den1 + 1e-16) + bg1[...]
    num2 = n2[0] + n2[1]
    den2 = jnp.dot(d2[0] + d2[1], Rm[...], preferred_element_type=f32)
    w2 = num2 / (den2 + 1e-16) + bg2[...]

    fx = fea[...]
    nrm = jnp.maximum(jnp.sqrt(jnp.sum(fx * fx, axis=-1, keepdims=True)), 1e-12)
    fn = fx / nrm
    w3 = jnp.maximum(jnp.dot(fn, Wf[...], preferred_element_type=f32) + bf[...], 0.0)

    W1a_v = W1a[...]
    e1 = jnp.maximum(
        jnp.dot(w1, W1a_v[:D], preferred_element_type=f32)
        + jnp.dot(w2, W1a_v[D:], preferred_element_type=f32) + b1a[...], 0.0)
    emb1 = jnp.dot(e1, W1b[...], preferred_element_type=f32) + b1b[...]
    W2a_v = W2a[...]
    e2 = jnp.maximum(
        jnp.dot(emb1, W2a_v[:D], preferred_element_type=f32)
        + jnp.dot(w3, W2a_v[D:], preferred_element_type=f32) + b2a[...], 0.0)
    emb_o[...] = jnp.dot(e2, W2b[...], preferred_element_type=f32) + b2b[...]


def _combine_mlp(n1, d1, n2, d2, fea, bg1, bg2, Wf, bf, W1a, b1a, W1b, b1b,
                 W2a, b2a, W2b, b2b, Rm):
    blk = 1000
    grid = (N // blk,)
    full = lambda shape: pl.BlockSpec(shape, lambda i: tuple(0 for _ in shape))
    part_n = pl.BlockSpec((NC, blk, D), lambda i: (0, i, 0))
    part_d = pl.BlockSpec((NC, blk, 16), lambda i: (0, i, 0))
    row = lambda w: pl.BlockSpec((blk, w), lambda i: (i, 0))
    return pl.pallas_call(
        _k3_body,
        grid=grid,
        in_specs=[part_n, part_d, part_n, part_d, row(FEA),
                  full((1, D)), full((1, D)), full((FEA, D)), full((1, D)),
                  full((2 * D, D)), full((1, D)), full((D, D)), full((1, D)),
                  full((2 * D, D)), full((1, D)), full((D, D)), full((1, D)),
                  full((16, D))],
        out_specs=row(D),
        out_shape=jax.ShapeDtypeStruct((N, D), f32),
    )(n1, d1, n2, d2, fea, bg1, bg2, Wf, bf, W1a, b1a, W1b, b1b,
      W2a, b2a, W2b, b2b, Rm)


def _batch_gather(emb, bidx_t):
    nrows = B * L
    per_w = nrows // NW
    mesh = plsc.VectorSubcoreMesh(core_axis_name="c", subcore_axis_name="s",
                                  num_cores=NC, num_subcores=NS)

    @functools.partial(
        pl.kernel,
        out_type=jax.ShapeDtypeStruct((nrows, D), f32),
        mesh=mesh,
        scratch_types=[
            pltpu.VMEM((per_w,), i32),
            pltpu.VMEM((per_w, D), f32),
            pltpu.SemaphoreType.DMA,
        ],
    )
    def k(emb_r, bidx_r, out_r, idxv, rows, sem):
        wid = lax.axis_index("c") * NS + lax.axis_index("s")
        base = wid * per_w
        pltpu.sync_copy(bidx_r.at[pl.ds(base, per_w)], idxv)
        pltpu.async_copy(emb_r.at[idxv], rows, sem).wait()
        pltpu.sync_copy(rows, out_r.at[pl.ds(base, per_w)])

    return k(emb, bidx_t)


def _sig(x):
    return 1.0 / (1.0 + jnp.exp(-x))


def _k5_body(xb, WihT0, WhhT0, bih0, bhh0, WihT1, WhhT1, bih1, bhh1,
             out_o, gx_ref, ys_ref, hc2_ref):
    xb_v = xb[...]

    def run_layer(x_flat, WihT, WhhT, bias):
        gx = jnp.dot(x_flat, WihT, preferred_element_type=f32) + bias
        gx_ref[...] = gx.reshape(L, B, 4 * D)
        z = jnp.zeros((B, D), f32)

        def step(t, carry):
            h, c = carry
            g = gx_ref[t] + jnp.dot(h, WhhT, preferred_element_type=f32)
            ii = _sig(g[:, :D])
            ff = _sig(g[:, D:2 * D])
            gg = jnp.tanh(g[:, 2 * D:3 * D])
            oo = _sig(g[:, 3 * D:])
            c2 = ff * c + ii * gg
            h2 = oo * jnp.tanh(c2)
            ys_ref[t] = h2
            return (h2, c2)

        lax.fori_loop(0, L, step, (z, z))

    def softmax_stats():
        def mx(t, m):
            return jnp.maximum(m, ys_ref[t])
        m = lax.fori_loop(0, L, mx, jnp.full((B, D), -jnp.inf, f32))

        def sm(t, s):
            return s + jnp.exp(ys_ref[t] - m)
        s = lax.fori_loop(0, L, sm, jnp.zeros((B, D), f32))
        return m, s

    run_layer(xb_v, WihT0[...], WhhT0[...], bih0[...] + bhh0[...])
    m, s = softmax_stats()

    def apply1(t, carry):
        yt = ys_ref[t]
        at = jnp.mean(jnp.exp(yt - m) / s, axis=-1, keepdims=True)
        o1 = yt * (1.0 + at)
        hc2_ref[t] = xb[t] + jnp.maximum(o1, 0.0)
        return carry

    lax.fori_loop(0, L, apply1, 0)

    run_layer(hc2_ref[...].reshape(L * B, D), WihT1[...], WhhT1[...],
              bih1[...] + bhh1[...])
    m2, s2 = softmax_stats()
    y_last = ys_ref[L - 1]
    a_last = jnp.mean(jnp.exp(y_last - m2) / s2, axis=-1, keepdims=True)
    out_o[...] = y_last * (1.0 + a_last)


def _lstm_head(xb, WihT0, WhhT0, bih0, bhh0, WihT1, WhhT1, bih1, bhh1):
    xb3 = xb.reshape(L, B, D)
    vm = pl.BlockSpec(memory_space=pltpu.MemorySpace.VMEM)
    return pl.pallas_call(
        _k5_body,
        in_specs=[vm] * 9,
        out_specs=vm,
        out_shape=jax.ShapeDtypeStruct((B, D), f32),
        scratch_shapes=[
            pltpu.VMEM((L, B, 4 * D), f32),
            pltpu.VMEM((L, B, D), f32),
            pltpu.VMEM((L, B, D), f32),
        ],
    )(xb3, WihT0, WhhT0, bih0, bhh0, WihT1, WhhT1, bih1, bhh1)


def kernel(poi_features, fea_x, edge_index, str_edge_index, batch_x,
           W_g1, a_src1, a_dst1, b_g1, W_g2, a_src2, a_dst2, b_g2,
           Wf, bf, W1a, b1a, W1b, b1b, W2a, b2a, W2b, b2b,
           Wih0, Whh0, bih0, bhh0, Wih1, Whh1, bih1, bhh1):
    Wcat = jnp.concatenate([W_g1, W_g2], axis=1)
    sel = jnp.tile(jnp.eye(H, dtype=f32), (1, 2))

    def mk(a):
        return (a[:, :, None] * sel[:, None, :]).reshape(H * C, 16)

    M1 = jnp.concatenate([mk(a_src1), mk(a_dst1)], axis=1)
    M2 = jnp.concatenate([mk(a_src2), mk(a_dst2)], axis=1)
    Rm = (jnp.arange(16)[:, None] == (jnp.arange(D) // C)[None, :]).astype(f32)
    zn = jnp.zeros((RBLK, D), f32)
    zd = jnp.zeros((RBLK, 16), f32)
    r1 = lambda b: b.reshape(1, -1)

    h1, h2, ps1, pd1, ps2, pd2 = _precompute(poi_features, Wcat, M1, M2)

    n1, d1, n2, d2 = _gat_edges(
        h1, ps1, pd1, h2, ps2, pd2,
        edge_index[0], edge_index[1],
        str_edge_index[0], str_edge_index[1], zn, zd)

    emb = _combine_mlp(n1, d1, n2, d2, fea_x, r1(b_g1), r1(b_g2), Wf, r1(bf),
                       W1a, r1(b1a), W1b, r1(b1b), W2a, r1(b2a), W2b, r1(b2b),
                       Rm)

    bidx_t = jnp.transpose(batch_x).reshape(-1).astype(i32)
    xb = _batch_gather(emb, bidx_t)

    return _lstm_head(xb, Wih0.T, Whh0.T, r1(bih0), r1(bhh0),
                      Wih1.T, Whh1.T, r1(bih1), r1(bhh1))


# speedup vs baseline: 1.2618x; 1.0458x over previous
import functools

import jax
import jax.numpy as jnp
from jax import lax
from jax.experimental import pallas as pl
from jax.experimental.pallas import tpu as pltpu
from jax.experimental.pallas import tpu_sc as plsc

N = 10000
D = 128
FEA = 32
H = 8
C = 16
E1 = 320000
E2 = 160000
B = 128
L = 32

NC = 2
NS = 16
NW = NC * NS
RBLK = 80
NRB = N // RBLK

f32 = jnp.float32
i32 = jnp.int32


def _k1_body(poi, Wcat, M1, M2, h1_o, h2_o, ps1_o, pd1_o, ps2_o, pd2_o):
    hcat = jnp.dot(poi[...], Wcat[...], preferred_element_type=f32)
    h1 = hcat[:, :D]
    h2 = hcat[:, D:]
    h1_o[...] = h1
    h2_o[...] = h2
    p1 = jnp.dot(h1, M1[...], preferred_element_type=f32)
    p2 = jnp.dot(h2, M2[...], preferred_element_type=f32)
    ps1_o[...] = p1[:, :16]
    pd1_o[...] = p1[:, 16:]
    ps2_o[...] = p2[:, :16]
    pd2_o[...] = p2[:, 16:]


def _precompute(poi, Wcat, M1, M2):
    blk = 1000
    grid = (N // blk,)
    full = lambda shape: pl.BlockSpec(shape, lambda i: (0, 0))
    row = lambda w: pl.BlockSpec((blk, w), lambda i: (i, 0))
    return pl.pallas_call(
        _k1_body,
        grid=grid,
        in_specs=[row(D), full((D, 2 * D)), full((D, 32)), full((D, 32))],
        out_specs=[row(D), row(D), row(16), row(16), row(16), row(16)],
        out_shape=[
            jax.ShapeDtypeStruct((N, D), f32),
            jax.ShapeDtypeStruct((N, D), f32),
            jax.ShapeDtypeStruct((N, 16), f32),
            jax.ShapeDtypeStruct((N, 16), f32),
            jax.ShapeDtypeStruct((N, 16), f32),
            jax.ShapeDtypeStruct((N, 16), f32),
        ],
    )(poi, Wcat, M1, M2)


def _lane_bcast(v, lane):
    idx = jnp.full((16, 1), lane, i32)
    dn = lax.GatherDimensionNumbers(
        offset_dims=(), collapsed_slice_dims=(0,), start_index_map=(0,))
    return lax.gather(v, idx, dn, (1,),
                      mode=lax.GatherScatterMode.PROMISE_IN_BOUNDS)


NBUF = 4


def _edge_pass(hmat, psm, pdm, srcr, dstr, zn, zd, accn, accd, num_o, den_o,
               bufs, gsems, ssems, isems, epw, K):
    cid = lax.axis_index("c")
    tid = lax.axis_index("s")
    wid = cid * NS + tid

    def for_my_row_blocks(fn):
        for i in range((NRB + NS - 1) // NS):
            blk = i * NS + tid

            @pl.when(blk < NRB)
            def _():
                fn(blk * RBLK)

    def zero_blk(off):
        pltpu.sync_copy(zn, accn.at[pl.ds(off, RBLK)])
        pltpu.sync_copy(zd, accd.at[pl.ds(off, RBLK)])

    for_my_row_blocks(zero_blk)
    plsc.subcore_barrier()

    nch = epw // K
    nepi = nch % NBUF

    def fire_idx(c, b):
        base = wid * epw + c * K
        pltpu.async_copy(srcr.at[pl.ds(base, K)], bufs[b][0], isems[b])
        pltpu.async_copy(dstr.at[pl.ds(base, K)], bufs[b][1], isems[b])

    def wait_idx(c, b):
        base = wid * epw + c * K
        pltpu.make_async_copy(srcr.at[pl.ds(base, K)], bufs[b][0], isems[b]).wait()
        pltpu.make_async_copy(dstr.at[pl.ds(base, K)], bufs[b][1], isems[b]).wait()

    def fire_gathers(b):
        sidx, didx, psb, pdb, hsb, dvb = bufs[b]
        pltpu.async_copy(psm.at[sidx], psb, gsems[b])
        pltpu.async_copy(pdm.at[didx], pdb, gsems[b])
        pltpu.async_copy(hmat.at[sidx], hsb, gsems[b])

    def wait_gathers(b):
        sidx, didx, psb, pdb, hsb, dvb = bufs[b]
        pltpu.make_async_copy(psm.at[sidx], psb, gsems[b]).wait()
        pltpu.make_async_copy(pdm.at[didx], pdb, gsems[b]).wait()
        pltpu.make_async_copy(hmat.at[sidx], hsb, gsems[b]).wait()

    def fire_scatters(b):
        sidx, didx, psb, pdb, hsb, dvb = bufs[b]
        pltpu.async_copy(hsb, accn.at[didx], ssems[b], add=True)
        pltpu.async_copy(dvb, accd.at[didx], ssems[b], add=True)

    def wait_scatters(b):
        sidx, didx, psb, pdb, hsb, dvb = bufs[b]
        pltpu.make_async_copy(hsb, accn.at[didx], ssems[b]).wait()
        pltpu.make_async_copy(dvb, accd.at[didx], ssems[b]).wait()

    def compute(b):
        sidx, didx, psb, pdb, hsb, dvb = bufs[b]

        @plsc.parallel_loop(0, K, unroll=4)
        def edge(k):
            e16 = psb[k, :] + pdb[k, :]
            e16 = jnp.maximum(e16, 0.2 * e16)
            ex = jnp.exp(e16)
            dvb[k, :] = ex
            for h in range(H):
                exh = _lane_bcast(ex, h)
                hsb[k, pl.ds(h * 16, 16)] = hsb[k, pl.ds(h * 16, 16)] * exh

    def step(c, j):
        b = j
        b1 = (j + 1) % NBUF
        b2 = (j + 2) % NBUF
        cond_pref = c + 2 < nch

        if isinstance(c, int):
            if cond_pref:
                if c >= 2:
                    wait_scatters(b2)
                fire_idx(c + 2, b2)
            if c + 1 < nch:
                wait_idx(c + 1, b1)
                fire_gathers(b1)
        else:
            @pl.when(cond_pref)
            def _():
                if j >= 2:
                    wait_scatters(b2)
                else:
                    @pl.when(c >= 2)
                    def _():
                        wait_scatters(b2)
                fire_idx(c + 2, b2)

            @pl.when(c + 1 < nch)
            def _():
                wait_idx(c + 1, b1)
                fire_gathers(b1)

        wait_gathers(b)
        compute(b)
        fire_scatters(b)

    fire_idx(0, 0)
    wait_idx(0, 0)
    fire_gathers(0)
    fire_idx(1, 1)

    def body(i, carry):
        for j in range(NBUF):
            step(NBUF * i + j, j)
        return carry

    lax.fori_loop(0, nch // NBUF, body, 0)

    for ce in range(nch - nepi, nch):
        step(ce, ce % NBUF)
    for b in range(NBUF):
        wait_scatters(b)
    plsc.subcore_barrier()

    def copyout_blk(off):
        pltpu.sync_copy(accn.at[pl.ds(off, RBLK)],
                        num_o.at[cid, pl.ds(off, RBLK)])
        pltpu.sync_copy(accd.at[pl.ds(off, RBLK)],
                        den_o.at[cid, pl.ds(off, RBLK)])

    for_my_row_blocks(copyout_blk)


KE = 40


def _gat_edges(h1, ps1, pd1, h2, ps2, pd2, src1, dst1, src2, dst2, zn, zd):
    mesh = plsc.VectorSubcoreMesh(core_axis_name="c", subcore_axis_name="s",
                                  num_cores=NC, num_subcores=NS)

    @functools.partial(
        pl.kernel,
        out_type=[
            jax.ShapeDtypeStruct((NC, N, D), f32),
            jax.ShapeDtypeStruct((NC, N, 16), f32),
            jax.ShapeDtypeStruct((NC, N, D), f32),
            jax.ShapeDtypeStruct((NC, N, 16), f32),
        ],
        mesh=mesh,
        scratch_types=(
            [pltpu.VMEM_SHARED((N, D), f32), pltpu.VMEM_SHARED((N, 16), f32)]
            + [pltpu.VMEM((KE,), i32), pltpu.VMEM((KE,), i32),
               pltpu.VMEM((KE, 16), f32), pltpu.VMEM((KE, 16), f32),
               pltpu.VMEM((KE, D), f32), pltpu.VMEM((KE, 16), f32)] * NBUF
            + [pltpu.SemaphoreType.DMA] * (3 * NBUF)
        ),
        compiler_params=pltpu.CompilerParams(use_tc_tiling_on_sc=False),
    )
    def k(h1r, ps1r, pd1r, h2r, ps2r, pd2r, s1r, d1r, s2r, d2r, znr, zdr,
          n1_o, d1_o, n2_o, d2_o, accn, accd, *scr):
        bufs = [tuple(scr[6 * b:6 * b + 6]) for b in range(NBUF)]
        sems = scr[6 * NBUF:]
        gsems = sems[:NBUF]
        ssems = sems[NBUF:2 * NBUF]
        isems = sems[2 * NBUF:]
        _edge_pass(h1r, ps1r, pd1r, s1r, d1r, znr, zdr, accn, accd,
                   n1_o, d1_o, bufs, gsems, ssems, isems, E1 // NW, KE)
        _edge_pass(h2r, ps2r, pd2r, s2r, d2r, znr, zdr, accn, accd,
                   n2_o, d2_o, bufs, gsems, ssems, isems, E2 // NW, KE)

    return k(h1, ps1, pd1, h2, ps2, pd2, src1, dst1, src2, dst2, zn, zd)


def _k3_body(n1, d1, n2, d2, fea, bg1, bg2, Wf, bf, W1a, b1a, W1b, b1b,
             W2a, b2a, W2b, b2b, Rm, emb_o):
    num1 = n1[0] + n1[1]
    den1 = jnp.dot(d1[0] + d1[1], Rm[...], preferred_element_type=f32)
    w1 = num1 / (den1 + 1e-16) + bg1[...]
    num2 = n2[0] + n2[1]
    den2 = jnp.dot(d2[0] + d2[1], Rm[...], preferred_element_type=f32)
    w2 = num2 / (den2 + 1e-16) + bg2[...]

    fx = fea[...]
    nrm = jnp.maximum(jnp.sqrt(jnp.sum(fx * fx, axis=-1, keepdims=True)), 1e-12)
    fn = fx / nrm
    w3 = jnp.maximum(jnp.dot(fn, Wf[...], preferred_element_type=f32) + bf[...], 0.0)

    W1a_v = W1a[...]
    e1 = jnp.maximum(
        jnp.dot(w1, W1a_v[:D], preferred_element_type=f32)
        + jnp.dot(w2, W1a_v[D:], preferred_element_type=f32) + b1a[...], 0.0)
    emb1 = jnp.dot(e1, W1b[...], preferred_element_type=f32) + b1b[...]
    W2a_v = W2a[...]
    e2 = jnp.maximum(
        jnp.dot(emb1, W2a_v[:D], preferred_element_type=f32)
        + jnp.dot(w3, W2a_v[D:], preferred_element_type=f32) + b2a[...], 0.0)
    emb_o[...] = jnp.dot(e2, W2b[...], preferred_element_type=f32) + b2b[...]


def _combine_mlp(n1, d1, n2, d2, fea, bg1, bg2, Wf, bf, W1a, b1a, W1b, b1b,
                 W2a, b2a, W2b, b2b, Rm):
    blk = 1000
    grid = (N // blk,)
    full = lambda shape: pl.BlockSpec(shape, lambda i: tuple(0 for _ in shape))
    part_n = pl.BlockSpec((NC, blk, D), lambda i: (0, i, 0))
    part_d = pl.BlockSpec((NC, blk, 16), lambda i: (0, i, 0))
    row = lambda w: pl.BlockSpec((blk, w), lambda i: (i, 0))
    return pl.pallas_call(
        _k3_body,
        grid=grid,
        in_specs=[part_n, part_d, part_n, part_d, row(FEA),
                  full((1, D)), full((1, D)), full((FEA, D)), full((1, D)),
                  full((2 * D, D)), full((1, D)), full((D, D)), full((1, D)),
                  full((2 * D, D)), full((1, D)), full((D, D)), full((1, D)),
                  full((16, D))],
        out_specs=row(D),
        out_shape=jax.ShapeDtypeStruct((N, D), f32),
    )(n1, d1, n2, d2, fea, bg1, bg2, Wf, bf, W1a, b1a, W1b, b1b,
      W2a, b2a, W2b, b2b, Rm)


def _batch_gather(emb, bidx_t):
    nrows = B * L
    per_w = nrows // NW
    mesh = plsc.VectorSubcoreMesh(core_axis_name="c", subcore_axis_name="s",
                                  num_cores=NC, num_subcores=NS)

    @functools.partial(
        pl.kernel,
        out_type=jax.ShapeDtypeStruct((nrows, D), f32),
        mesh=mesh,
        scratch_types=[
            pltpu.VMEM((per_w,), i32),
            pltpu.VMEM((per_w, D), f32),
            pltpu.SemaphoreType.DMA,
        ],
    )
    def k(emb_r, bidx_r, out_r, idxv, rows, sem):
        wid = lax.axis_index("c") * NS + lax.axis_index("s")
        base = wid * per_w
        pltpu.sync_copy(bidx_r.at[pl.ds(base, per_w)], idxv)
        pltpu.async_copy(emb_r.at[idxv], rows, sem).wait()
        pltpu.sync_copy(rows, out_r.at[pl.ds(base, per_w)])

    return k(emb, bidx_t)


def _sig(x):
    return 0.5 + 0.5 * jnp.tanh(0.5 * x)


def _k5_body(xb, WihT0, WhhT0, bih0, bhh0, WihT1, WhhT1, bih1, bhh1,
             out_o, gx_ref, ys_ref, hc2_ref):
    xb_v = xb[...]

    def run_layer(x_flat, WihT, WhhT, bias):
        gx = jnp.dot(x_flat, WihT, preferred_element_type=f32) + bias
        gx_ref[...] = gx.reshape(L, B, 4 * D)
        z = jnp.zeros((B, D), f32)

        def step(t, carry):
            h, c = carry
            g = gx_ref[t] + jnp.dot(h, WhhT, preferred_element_type=f32)
            ii = _sig(g[:, :D])
            ff = _sig(g[:, D:2 * D])
            gg = jnp.tanh(g[:, 2 * D:3 * D])
            oo = _sig(g[:, 3 * D:])
            c2 = ff * c + ii * gg
            h2 = oo * jnp.tanh(c2)
            ys_ref[t] = h2
            return (h2, c2)

        lax.fori_loop(0, L, step, (z, z))

    def softmax_denom():
        def sm(t, s):
            return s + jnp.exp(ys_ref[t])
        return lax.fori_loop(0, L, sm, jnp.zeros((B, D), f32))

    run_layer(xb_v, WihT0[...], WhhT0[...], bih0[...] + bhh0[...])
    s = softmax_denom()

    def apply1(t, carry):
        yt = ys_ref[t]
        at = jnp.mean(jnp.exp(yt) / s, axis=-1, keepdims=True)
        o1 = yt * (1.0 + at)
        hc2_ref[t] = xb[t] + jnp.maximum(o1, 0.0)
        return carry

    lax.fori_loop(0, L, apply1, 0)

    run_layer(hc2_ref[...].reshape(L * B, D), WihT1[...], WhhT1[...],
              bih1[...] + bhh1[...])
    s2 = softmax_denom()
    y_last = ys_ref[L - 1]
    a_last = jnp.mean(jnp.exp(y_last) / s2, axis=-1, keepdims=True)
    out_o[...] = y_last * (1.0 + a_last)


def _lstm_head(xb, WihT0, WhhT0, bih0, bhh0, WihT1, WhhT1, bih1, bhh1):
    xb3 = xb.reshape(L, B, D)
    vm = pl.BlockSpec(memory_space=pltpu.MemorySpace.VMEM)
    return pl.pallas_call(
        _k5_body,
        in_specs=[vm] * 9,
        out_specs=vm,
        out_shape=jax.ShapeDtypeStruct((B, D), f32),
        scratch_shapes=[
            pltpu.VMEM((L, B, 4 * D), f32),
            pltpu.VMEM((L, B, D), f32),
            pltpu.VMEM((L, B, D), f32),
        ],
    )(xb3, WihT0, WhhT0, bih0, bhh0, WihT1, WhhT1, bih1, bhh1)


def kernel(poi_features, fea_x, edge_index, str_edge_index, batch_x,
           W_g1, a_src1, a_dst1, b_g1, W_g2, a_src2, a_dst2, b_g2,
           Wf, bf, W1a, b1a, W1b, b1b, W2a, b2a, W2b, b2b,
           Wih0, Whh0, bih0, bhh0, Wih1, Whh1, bih1, bhh1):
    Wcat = jnp.concatenate([W_g1, W_g2], axis=1)
    sel = jnp.tile(jnp.eye(H, dtype=f32), (1, 2))

    def mk(a):
        return (a[:, :, None] * sel[:, None, :]).reshape(H * C, 16)

    M1 = jnp.concatenate([mk(a_src1), mk(a_dst1)], axis=1)
    M2 = jnp.concatenate([mk(a_src2), mk(a_dst2)], axis=1)
    Rm = (jnp.arange(16)[:, None] == (jnp.arange(D) // C)[None, :]).astype(f32)
    zn = jnp.zeros((RBLK, D), f32)
    zd = jnp.zeros((RBLK, 16), f32)
    r1 = lambda b: b.reshape(1, -1)

    h1, h2, ps1, pd1, ps2, pd2 = _precompute(poi_features, Wcat, M1, M2)

    n1, d1, n2, d2 = _gat_edges(
        h1, ps1, pd1, h2, ps2, pd2,
        edge_index[0], edge_index[1],
        str_edge_index[0], str_edge_index[1], zn, zd)

    emb = _combine_mlp(n1, d1, n2, d2, fea_x, r1(b_g1), r1(b_g2), Wf, r1(bf),
                       W1a, r1(b1a), W1b, r1(b1b), W2a, r1(b2a), W2b, r1(b2b),
                       Rm)

    bidx_t = jnp.transpose(batch_x).reshape(-1).astype(i32)
    xb = _batch_gather(emb, bidx_t)

    return _lstm_head(xb, Wih0.T, Whh0.T, r1(bih0), r1(bhh0),
                      Wih1.T, Whh1.T, r1(bih1), r1(bhh1))
